# baseline (speedup 1.0000x reference)
import functools

import jax
import jax.numpy as jnp
from jax.experimental import pallas as pl
from jax.experimental.pallas import tpu as pltpu

_STRENGTH_DECAY = 0.995
_AGE_PENALTY = 0.02
_STRENGTH_BOOST = 0.5
_WRITE_ALPHA = 0.5
_WRITE_TEMP = 50.0
_EVICT_AGE_BOOST = 0.05
_EVICT_STR_PENALTY = 0.5


def _bf16_val(x):
    return x.astype(jnp.bfloat16).astype(jnp.float32)


def _split_hi_lo(p):
    i = jax.lax.bitcast_convert_type(p, jnp.uint32)
    hi = jax.lax.bitcast_convert_type(i & jnp.uint32(0xFFFF0000), jnp.float32)
    lo = p - hi
    return hi.astype(jnp.bfloat16), lo.astype(jnp.bfloat16)


def _expand_slots(x, reps):
    K = x.shape[1]
    return jnp.concatenate(
        [jnp.broadcast_to(x[:, k:k + 1], (x.shape[0], reps))
         for k in range(K)], axis=1)


def _block_body(K, D, r_ref, q_ref, w_ref, ws_ref, keys_ref, vals_ref,
                age_ref, str_ref,
                read_ref, kout_ref, vout_ref, ageout_ref, strout_ref):
    R = r_ref[...]
    q = q_ref[...]
    wv = w_ref[...]
    ws = ws_ref[...]
    keys = keys_ref[...]
    vals = vals_ref[...]
    age = age_ref[...]
    stg = str_ref[...]
    bB = q.shape[0]

    dot = functools.partial(jnp.dot, preferred_element_type=jnp.float32)

    q_den = jnp.sqrt(jnp.sum(q * q, axis=-1, keepdims=True)) + 1e-6
    w_den = jnp.sqrt(jnp.sum(wv * wv, axis=-1, keepdims=True)) + 1e-6
    qn_b = _bf16_val(q / q_den)
    wk_n = wv / w_den
    wn_b = _bf16_val(wk_n)

    sq_hi, sq_lo = _split_hi_lo(keys * keys)
    nsq = dot(jnp.concatenate([sq_hi, sq_lo], axis=0), R)
    k_den = jnp.sqrt(nsq[:bB] + nsq[bB:]) + 1e-6

    kn = keys * _expand_slots(1.0 / k_den, D)
    kn_f = _bf16_val(kn)

    q_rep = pltpu.repeat(qn_b, K, axis=1)
    w_rep = pltpu.repeat(wn_b, K, axis=1)
    pq_hi, pq_lo = _split_hi_lo(kn_f * q_rep)
    pw_hi, pw_lo = _split_hi_lo(kn_f * w_rep)
    sims = dot(jnp.concatenate([pq_hi, pq_lo, pw_hi, pw_lo], axis=0), R)
    sim_r = sims[:bB] + sims[bB:2 * bB]
    sim_w = sims[2 * bB:3 * bB] + sims[3 * bB:]

    logits = (sim_r + _STRENGTH_BOOST * jnp.log(jnp.clip(stg, 0.001, 1.0))
              - _AGE_PENALTY * age)
    m_r = jnp.max(logits, axis=-1, keepdims=True)
    e_r = jnp.exp(logits - m_r)
    w_read = e_r / jnp.sum(e_r, axis=-1, keepdims=True)

    p_r = _bf16_val(vals) * _expand_slots(_bf16_val(w_read), D)
    width = K * D
    while width > D:
        width //= 2
        p_r = p_r[:, :width] + p_r[:, width:2 * width]
    read_ref[...] = p_r

    wl = (sim_w * _WRITE_TEMP + _EVICT_AGE_BOOST * jnp.log1p(age)
          - _EVICT_STR_PENALTY * stg)
    m_w = jnp.max(wl, axis=-1, keepdims=True)
    e_w = jnp.exp(wl - m_w)
    soft = e_w / jnp.sum(e_w, axis=-1, keepdims=True)

    iota = jax.lax.broadcasted_iota(jnp.int32, soft.shape, 1)
    s_max = jnp.max(soft, axis=-1, keepdims=True)
    top = jnp.min(jnp.where(soft == s_max, iota, K), axis=-1, keepdims=True)
    hard = (iota == top).astype(jnp.float32)
    write_w = (hard - soft) + soft

    wsc = jnp.clip(ws, 0.0, 1.0)
    eff = write_w * wsc * _WRITE_ALPHA
    rate = _expand_slots(eff, D)
    one_m = 1.0 - rate
    kout_ref[...] = one_m * keys + rate * pltpu.repeat(wk_n, K, axis=1)
    vout_ref[...] = one_m * vals + rate * pltpu.repeat(wv, K, axis=1)
    ageout_ref[...] = (age + 1.0) * (1.0 - write_w)
    s_new = stg * _STRENGTH_DECAY + write_w * wsc * (1.0 - stg * _STRENGTH_DECAY)
    strout_ref[...] = jnp.clip(s_new, 0.0, 1.0)


def kernel(query_vec, write_vec, write_strength, keys, vals, age, strength,
           *, block_rows=128, interpret=False):
    B, D = query_vec.shape
    K = keys.shape[1]
    KD = K * D
    bB = block_rows if B % block_rows == 0 else 8
    grid = (B // bB,)

    keys2 = keys.reshape(B, KD)
    vals2 = vals.reshape(B, KD)
    rhs = jnp.repeat(jnp.eye(K, dtype=jnp.bfloat16), D, axis=0)

    row2 = lambda i: (i, 0)
    f32 = jnp.float32

    out_shapes = (
        jax.ShapeDtypeStruct((B, D), f32),
        jax.ShapeDtypeStruct((B, KD), f32),
        jax.ShapeDtypeStruct((B, KD), f32),
        jax.ShapeDtypeStruct((B, K), f32),
        jax.ShapeDtypeStruct((B, K), f32),
    )
    out_specs = (
        pl.BlockSpec((bB, D), row2),
        pl.BlockSpec((bB, KD), row2),
        pl.BlockSpec((bB, KD), row2),
        pl.BlockSpec((bB, K), row2),
        pl.BlockSpec((bB, K), row2),
    )
    in_specs = [
        pl.BlockSpec((KD, K), lambda i: (0, 0)),
        pl.BlockSpec((bB, D), row2),
        pl.BlockSpec((bB, D), row2),
        pl.BlockSpec((bB, 1), row2),
        pl.BlockSpec((bB, KD), row2),
        pl.BlockSpec((bB, KD), row2),
        pl.BlockSpec((bB, K), row2),
        pl.BlockSpec((bB, K), row2),
    ]

    read_out, keys_new, vals_new, age_new, str_new = pl.pallas_call(
        functools.partial(_block_body, K, D),
        out_shape=out_shapes,
        grid=grid,
        in_specs=in_specs,
        out_specs=out_specs,
        compiler_params=pltpu.CompilerParams(
            dimension_semantics=("parallel",),
            vmem_limit_bytes=56 * 1024 * 1024,
        ),
        name="episodic_slot_memory",
        interpret=interpret,
    )(rhs, query_vec, write_vec, write_strength, keys2, vals2, age, strength)

    return (read_out, (keys_new.reshape(B, K, D), vals_new.reshape(B, K, D),
                       age_new, str_new))


# speedup vs baseline: 1.7383x; 1.7383x over previous
import functools

import jax
import jax.numpy as jnp
from jax.experimental import pallas as pl
from jax.experimental.pallas import tpu as pltpu

_STRENGTH_DECAY = 0.995
_AGE_PENALTY = 0.02
_STRENGTH_BOOST = 0.5
_WRITE_ALPHA = 0.5
_WRITE_TEMP = 50.0
_EVICT_AGE_BOOST = 0.05
_EVICT_STR_PENALTY = 0.5


def _bf16_val(x):
    return x.astype(jnp.bfloat16).astype(jnp.float32)


def _block_body(q_ref, w_ref, ws_ref, keys_ref, vals_ref, age_ref, str_ref,
                read_ref, kout_ref, vout_ref, ageout_ref, strout_ref):
    q = q_ref[...]
    wv = w_ref[...]
    ws = ws_ref[...]
    keys = keys_ref[...]
    vals = vals_ref[...]
    age = age_ref[...]
    stg = str_ref[...]
    K = keys.shape[1]

    q_den = jnp.sqrt(jnp.sum(q * q, axis=-1, keepdims=True)) + 1e-6
    w_den = jnp.sqrt(jnp.sum(wv * wv, axis=-1, keepdims=True)) + 1e-6
    k_den = jnp.sqrt(jnp.sum(keys * keys, axis=-1, keepdims=True)) + 1e-6
    qn = q / q_den
    wk_n = wv / w_den
    kn_b = _bf16_val(keys / k_den)
    qn_b = _bf16_val(qn)
    wn_b = _bf16_val(wk_n)

    sim_r = jnp.sum(kn_b * qn_b[:, None, :], axis=-1)
    sim_w = jnp.sum(kn_b * wn_b[:, None, :], axis=-1)

    logits = (sim_r + _STRENGTH_BOOST * jnp.log(jnp.clip(stg, 0.001, 1.0))
              - _AGE_PENALTY * age)
    m_r = jnp.max(logits, axis=-1, keepdims=True)
    e_r = jnp.exp(logits - m_r)
    w_read = e_r / jnp.sum(e_r, axis=-1, keepdims=True)
    read_ref[...] = jnp.sum(
        _bf16_val(w_read)[:, :, None] * _bf16_val(vals), axis=1)

    wl = (sim_w * _WRITE_TEMP + _EVICT_AGE_BOOST * jnp.log1p(age)
          - _EVICT_STR_PENALTY * stg)
    m_w = jnp.max(wl, axis=-1, keepdims=True)
    e_w = jnp.exp(wl - m_w)
    soft = e_w / jnp.sum(e_w, axis=-1, keepdims=True)

    iota = jax.lax.broadcasted_iota(jnp.int32, soft.shape, 1)
    s_max = jnp.max(soft, axis=-1, keepdims=True)
    top = jnp.min(jnp.where(soft == s_max, iota, K), axis=-1, keepdims=True)
    hard = (iota == top).astype(jnp.float32)
    write_w = (hard - soft) + soft

    wsc = jnp.clip(ws, 0.0, 1.0)
    eff = write_w * wsc * _WRITE_ALPHA
    rate = eff[:, :, None]
    kout_ref[...] = (1.0 - rate) * keys + rate * wk_n[:, None, :]
    vout_ref[...] = (1.0 - rate) * vals + rate * wv[:, None, :]
    ageout_ref[...] = (age + 1.0) * (1.0 - write_w)
    s_new = stg * _STRENGTH_DECAY + write_w * wsc * (1.0 - stg * _STRENGTH_DECAY)
    strout_ref[...] = jnp.clip(s_new, 0.0, 1.0)


def kernel(query_vec, write_vec, write_strength, keys, vals, age, strength,
           *, block_rows=128, semantics="parallel", interpret=False):
    B, D = query_vec.shape
    K = keys.shape[1]
    bB = block_rows if B % block_rows == 0 else 8
    grid = (B // bB,)

    row2 = lambda i: (i, 0)
    row3 = lambda i: (i, 0, 0)
    f32 = jnp.float32

    out_shapes = (
        jax.ShapeDtypeStruct((B, D), f32),
        jax.ShapeDtypeStruct((B, K, D), f32),
        jax.ShapeDtypeStruct((B, K, D), f32),
        jax.ShapeDtypeStruct((B, K), f32),
        jax.ShapeDtypeStruct((B, K), f32),
    )
    out_specs = (
        pl.BlockSpec((bB, D), row2),
        pl.BlockSpec((bB, K, D), row3),
        pl.BlockSpec((bB, K, D), row3),
        pl.BlockSpec((bB, K), row2),
        pl.BlockSpec((bB, K), row2),
    )
    in_specs = [
        pl.BlockSpec((bB, D), row2),
        pl.BlockSpec((bB, D), row2),
        pl.BlockSpec((bB, 1), row2),
        pl.BlockSpec((bB, K, D), row3),
        pl.BlockSpec((bB, K, D), row3),
        pl.BlockSpec((bB, K), row2),
        pl.BlockSpec((bB, K), row2),
    ]

    read_out, keys_new, vals_new, age_new, str_new = pl.pallas_call(
        _block_body,
        out_shape=out_shapes,
        grid=grid,
        in_specs=in_specs,
        out_specs=out_specs,
        compiler_params=pltpu.CompilerParams(
            dimension_semantics=(semantics,),
            vmem_limit_bytes=56 * 1024 * 1024,
        ),
        name="episodic_slot_memory",
        interpret=interpret,
    )(query_vec, write_vec, write_strength, keys, vals, age, strength)

    return (read_out, (keys_new, vals_new, age_new, str_new))


# speedup vs baseline: 2.9481x; 1.6960x over previous
import functools

import jax
import jax.numpy as jnp
from jax.experimental import pallas as pl
from jax.experimental.pallas import tpu as pltpu

_STRENGTH_DECAY = 0.995
_AGE_PENALTY = 0.02
_STRENGTH_BOOST = 0.5
_WRITE_ALPHA = 0.5
_WRITE_TEMP = 50.0
_EVICT_AGE_BOOST = 0.05
_EVICT_STR_PENALTY = 0.5


def _bf16_val(x):
    return x.astype(jnp.bfloat16).astype(jnp.float32)


def _block_body(q_ref, w_ref, ws_ref, keys_ref, vals_ref, age_ref, str_ref,
                read_ref, kout_ref, vout_ref, ageout_ref, strout_ref):
    q = q_ref[...]
    wv = w_ref[...]
    ws = ws_ref[...]
    age = age_ref[...]
    stg = str_ref[...]
    K = keys_ref.shape[1]

    q_den = jnp.sqrt(jnp.sum(q * q, axis=-1, keepdims=True)) + 1e-6
    w_den = jnp.sqrt(jnp.sum(wv * wv, axis=-1, keepdims=True)) + 1e-6
    qn = q / q_den
    wk_n = wv / w_den
    qn_b = _bf16_val(qn)
    wn_b = _bf16_val(wk_n)

    CH = 8
    sims_r, sims_w = [], []
    for c in range(0, K, CH):
        kc = keys_ref[:, c:c + CH, :]
        nc = jnp.sqrt(jnp.sum(kc * kc, axis=-1, keepdims=True)) + 1e-6
        knc = _bf16_val(kc / nc)
        sims_r.append(jnp.sum(knc * qn_b[:, None, :], axis=-1))
        sims_w.append(jnp.sum(knc * wn_b[:, None, :], axis=-1))
    sim_r = jnp.concatenate(sims_r, axis=1)
    sim_w = jnp.concatenate(sims_w, axis=1)

    logits = (sim_r + _STRENGTH_BOOST * jnp.log(jnp.clip(stg, 0.001, 1.0))
              - _AGE_PENALTY * age)
    m_r = jnp.max(logits, axis=-1, keepdims=True)
    e_r = jnp.exp(logits - m_r)
    w_read = e_r / jnp.sum(e_r, axis=-1, keepdims=True)
    w_read_b = _bf16_val(w_read)

    wl = (sim_w * _WRITE_TEMP + _EVICT_AGE_BOOST * jnp.log1p(age)
          - _EVICT_STR_PENALTY * stg)
    m_w = jnp.max(wl, axis=-1, keepdims=True)
    e_w = jnp.exp(wl - m_w)
    soft = e_w / jnp.sum(e_w, axis=-1, keepdims=True)

    iota = jax.lax.broadcasted_iota(jnp.int32, soft.shape, 1)
    s_max = jnp.max(soft, axis=-1, keepdims=True)
    top = jnp.min(jnp.where(soft == s_max, iota, K), axis=-1, keepdims=True)
    hard = (iota == top).astype(jnp.float32)
    write_w = (hard - soft) + soft

    wsc = jnp.clip(ws, 0.0, 1.0)
    eff = write_w * wsc * _WRITE_ALPHA
    racc = None
    for c in range(0, K, CH):
        kc = keys_ref[:, c:c + CH, :]
        vc = vals_ref[:, c:c + CH, :]
        rate_c = eff[:, c:c + CH, None]
        kout_ref[:, c:c + CH, :] = (1.0 - rate_c) * kc + rate_c * wk_n[:, None, :]
        vout_ref[:, c:c + CH, :] = (1.0 - rate_c) * vc + rate_c * wv[:, None, :]
        part = jnp.sum(w_read_b[:, c:c + CH, None] * _bf16_val(vc), axis=1)
        racc = part if racc is None else racc + part
    read_ref[...] = racc
    ageout_ref[...] = (age + 1.0) * (1.0 - write_w)
    s_new = stg * _STRENGTH_DECAY + write_w * wsc * (1.0 - stg * _STRENGTH_DECAY)
    strout_ref[...] = jnp.clip(s_new, 0.0, 1.0)


def kernel(query_vec, write_vec, write_strength, keys, vals, age, strength,
           *, block_rows=128, semantics="parallel", interpret=False):
    B, D = query_vec.shape
    K = keys.shape[1]
    bB = block_rows if B % block_rows == 0 else 8
    grid = (B // bB,)

    row2 = lambda i: (i, 0)
    row3 = lambda i: (i, 0, 0)
    f32 = jnp.float32

    out_shapes = (
        jax.ShapeDtypeStruct((B, D), f32),
        jax.ShapeDtypeStruct((B, K, D), f32),
        jax.ShapeDtypeStruct((B, K, D), f32),
        jax.ShapeDtypeStruct((B, K), f32),
        jax.ShapeDtypeStruct((B, K), f32),
    )
    out_specs = (
        pl.BlockSpec((bB, D), row2),
        pl.BlockSpec((bB, K, D), row3),
        pl.BlockSpec((bB, K, D), row3),
        pl.BlockSpec((bB, K), row2),
        pl.BlockSpec((bB, K), row2),
    )
    in_specs = [
        pl.BlockSpec((bB, D), row2),
        pl.BlockSpec((bB, D), row2),
        pl.BlockSpec((bB, 1), row2),
        pl.BlockSpec((bB, K, D), row3),
        pl.BlockSpec((bB, K, D), row3),
        pl.BlockSpec((bB, K), row2),
        pl.BlockSpec((bB, K), row2),
    ]

    read_out, keys_new, vals_new, age_new, str_new = pl.pallas_call(
        _block_body,
        out_shape=out_shapes,
        grid=grid,
        in_specs=in_specs,
        out_specs=out_specs,
        compiler_params=pltpu.CompilerParams(
            dimension_semantics=(semantics,),
            vmem_limit_bytes=56 * 1024 * 1024,
        ),
        name="episodic_slot_memory",
        interpret=interpret,
    )(query_vec, write_vec, write_strength, keys, vals, age, strength)

    return (read_out, (keys_new, vals_new, age_new, str_new))
